# baseline (speedup 1.0000x reference)
import jax
import jax.numpy as jnp
from jax import lax
from jax.experimental import pallas as pl
from jax.experimental.pallas import tpu as pltpu

N_BOX = 1000
N_PAD = 1024
INF = 1e9
IOU_T = 0.1


def _shift_lanes(v, k):
    return jnp.concatenate(
        [jnp.full((8, k), INF, jnp.float32), v[:, : 128 - k]], axis=1
    )


def _shift_subs(v, k):
    return jnp.concatenate(
        [jnp.full((k, 1), INF, jnp.float32), v[: 8 - k, :]], axis=0
    )


def _body(cv_ref, rep_ref, ii_ref, conf_ref, out_ref, adj_ref):
    cx, cy, w, h = cv_ref[0], cv_ref[1], cv_ref[2], cv_ref[3]
    X1 = cx - 0.5 * w
    Y1 = cy - 0.5 * h
    X2 = cx + 0.5 * w
    Y2 = cy + 0.5 * h
    AREA = w * h
    ROW = lax.broadcasted_iota(jnp.int32, (8, 128), 0)
    LANE = lax.broadcasted_iota(jnp.int32, (8, 128), 1)
    IDX = (ROW * 128 + LANE).astype(jnp.float32)
    CONF = conf_ref[:]

    def til(v):
        return jnp.broadcast_to(v[None], (64, 8, 128)).reshape(512, 128)

    X1T, Y1T, X2T, Y2T, AREAT, JT = (til(v) for v in (X1, Y1, X2, Y2, AREA, IDX))

    def ablock(bi, carry):
        base = bi * 512
        cxa = rep_ref[0, pl.ds(base, 512), :]
        cya = rep_ref[1, pl.ds(base, 512), :]
        wa = rep_ref[2, pl.ds(base, 512), :]
        ha = rep_ref[3, pl.ds(base, 512), :]
        iia = ii_ref[pl.ds(base, 512), :]
        x1a = cxa - 0.5 * wa
        y1a = cya - 0.5 * ha
        x2a = x1a + wa
        y2a = y1a + ha
        iw = jnp.maximum(jnp.minimum(x2a, X2T) - jnp.maximum(x1a, X1T), 0.0)
        ih = jnp.maximum(jnp.minimum(y2a, Y2T) - jnp.maximum(y1a, Y1T), 0.0)
        inter = iw * ih
        union = wa * ha + AREAT - inter
        adjb = (inter > IOU_T * union) & (JT > iia)
        val = jnp.where(JT == iia, 2.0, jnp.where(adjb, 1.0, 0.0))
        adj_ref[pl.ds(base, 512), :] = val
        return carry

    lax.fori_loop(0, 16, ablock, 0)

    def rowbody(i, assign):
        row = adj_ref[pl.ds(8 * i, 8), :]
        maskj = row == 1.0
        vfull = jnp.where(row >= 1.0, assign, INF)
        p = vfull
        for k in (1, 2, 4, 8, 16, 32, 64):
            p = jnp.minimum(p, _shift_lanes(p, k))
        rowtot = p[:, 127:128]
        t = _shift_subs(rowtot, 1)
        for k in (1, 2, 4):
            t = jnp.minimum(t, _shift_subs(t, k))
        p = jnp.minimum(p, t)
        totb = jnp.broadcast_to(
            jnp.minimum(rowtot[7:8, :], t[7:8, :]), (8, 128)
        )
        assign = jnp.where(maskj, p, assign)
        return jnp.where(row == 2.0, totb, assign)

    assign = lax.fori_loop(0, N_BOX, rowbody, IDX, unroll=False)

    def _r3(x, op):
        return op(op(x, axis=2, keepdims=True), axis=1, keepdims=True)

    def cblock(cb, maskacc):
        c0 = (cb * 128).astype(jnp.float32)
        C = lax.broadcasted_iota(jnp.int32, (128, 1, 1), 0).astype(jnp.float32) + c0
        M = assign[None, :, :] == C
        cnt = _r3(jnp.where(M, 1.0, 0.0), jnp.sum)
        mc = _r3(jnp.where(M, CONF[None], -INF), jnp.max)
        g = _r3(jnp.where(M & (CONF[None] == mc), IDX[None], INF), jnp.min)
        loc = _r3(jnp.where(M & (IDX[None] < g), 1.0, 0.0), jnp.sum)
        repr_ = jnp.where(cnt == 1.0, g, loc)
        valid = (cnt > 0.0) & (C < float(N_BOX))
        hits = (IDX[None] == repr_) & valid
        contrib = jnp.max(jnp.where(hits, 1.0, 0.0), axis=0)
        return jnp.maximum(maskacc, contrib)

    maskacc = lax.fori_loop(0, 8, cblock, jnp.zeros((8, 128), jnp.float32))
    out_ref[:, :] = maskacc


def kernel(bboxes_cxcywh, conf):
    coords = jnp.transpose(bboxes_cxcywh).astype(jnp.float32)
    coords = jnp.pad(coords, ((0, 0), (0, N_PAD - N_BOX)))
    coords_vec = coords.reshape(4, 8, 128)
    rep = jnp.repeat(coords, 8, axis=1).reshape(4, 8 * N_PAD, 1)
    ii = jnp.repeat(
        jnp.arange(N_PAD, dtype=jnp.float32), 8
    ).reshape(8 * N_PAD, 1)
    confp = jnp.pad(conf.astype(jnp.float32), (0, N_PAD - N_BOX)).reshape(8, 128)
    out = pl.pallas_call(
        _body,
        in_specs=[
            pl.BlockSpec(memory_space=pltpu.VMEM),
            pl.BlockSpec(memory_space=pltpu.VMEM),
            pl.BlockSpec(memory_space=pltpu.VMEM),
            pl.BlockSpec(memory_space=pltpu.VMEM),
        ],
        out_specs=pl.BlockSpec(memory_space=pltpu.VMEM),
        out_shape=jax.ShapeDtypeStruct((8, 128), jnp.float32),
        scratch_shapes=[pltpu.VMEM((8 * N_PAD, 128), jnp.float32)],
    )(coords_vec, rep, ii, confp)
    return out.reshape(N_PAD)[:N_BOX] > 0.5


# speedup vs baseline: 1.5065x; 1.5065x over previous
import jax
import jax.numpy as jnp
from jax import lax
from jax.experimental import pallas as pl
from jax.experimental.pallas import tpu as pltpu

N_BOX = 1000
N_PAD = 1024
INF = 1e9
IOU_T = 0.1


def _shift_lanes(v, k):
    return jnp.concatenate(
        [jnp.full((8, k), INF, jnp.float32), v[:, : 128 - k]], axis=1
    )


def _shift_subs(v, k):
    return jnp.concatenate(
        [jnp.full((k, 1), INF, jnp.float32), v[: 8 - k, :]], axis=0
    )


def _body(cv_ref, rep_ref, ii_ref, conf_ref, out_ref, adj_ref):
    cx, cy, w, h = cv_ref[0], cv_ref[1], cv_ref[2], cv_ref[3]
    X1 = cx - 0.5 * w
    Y1 = cy - 0.5 * h
    X2 = cx + 0.5 * w
    Y2 = cy + 0.5 * h
    AREA = w * h
    ROW = lax.broadcasted_iota(jnp.int32, (8, 128), 0)
    LANE = lax.broadcasted_iota(jnp.int32, (8, 128), 1)
    IDX = (ROW * 128 + LANE).astype(jnp.float32)
    CONF = conf_ref[:]

    def til(v):
        return jnp.broadcast_to(v[None], (64, 8, 128)).reshape(512, 128)

    X1T, Y1T, X2T, Y2T, AREAT, JT = (til(v) for v in (X1, Y1, X2, Y2, AREA, IDX))

    def ablock(bi, carry):
        base = bi * 512
        cxa = rep_ref[0, pl.ds(base, 512), :]
        cya = rep_ref[1, pl.ds(base, 512), :]
        wa = rep_ref[2, pl.ds(base, 512), :]
        ha = rep_ref[3, pl.ds(base, 512), :]
        iia = ii_ref[pl.ds(base, 512), :]
        x1a = cxa - 0.5 * wa
        y1a = cya - 0.5 * ha
        x2a = x1a + wa
        y2a = y1a + ha
        iw = jnp.maximum(jnp.minimum(x2a, X2T) - jnp.maximum(x1a, X1T), 0.0)
        ih = jnp.maximum(jnp.minimum(y2a, Y2T) - jnp.maximum(y1a, Y1T), 0.0)
        inter = iw * ih
        union = wa * ha + AREAT - inter
        adjb = (inter > IOU_T * union) & (JT > iia)
        val = jnp.where(JT == iia, 2.0, jnp.where(adjb, 1.0, 0.0))
        adj_ref[pl.ds(bi * 64, 64)] = val.reshape(64, 8, 128)
        return carry

    lax.fori_loop(0, 16, ablock, 0)

    def rowbody(i, assign):
        row = adj_ref[i]
        maskj = row == 1.0
        vfull = jnp.where(row >= 1.0, assign, INF)
        p = vfull
        for ks in ((1, 2, 3, 4, 5, 6, 7), (8, 16, 24, 32, 40, 48, 56), (64,)):
            vals = [p] + [_shift_lanes(p, k) for k in ks]
            while len(vals) > 1:
                vals = [
                    jnp.minimum(vals[j], vals[j + 1]) if j + 1 < len(vals) else vals[j]
                    for j in range(0, len(vals), 2)
                ]
            p = vals[0]
        rowtot = p[:, 127:128]
        t = _shift_subs(rowtot, 1)
        for k in (1, 2, 4):
            t = jnp.minimum(t, _shift_subs(t, k))
        p = jnp.minimum(p, t)
        totb = jnp.broadcast_to(
            jnp.minimum(rowtot[7:8, :], t[7:8, :]), (8, 128)
        )
        assign = jnp.where(maskj, p, assign)
        return jnp.where(row == 2.0, totb, assign)

    assign = lax.fori_loop(0, N_BOX, rowbody, IDX, unroll=False)

    def _r3(x, op):
        return op(op(x, axis=2, keepdims=True), axis=1, keepdims=True)

    def cblock(cb, maskacc):
        c0 = (cb * 128).astype(jnp.float32)
        C = lax.broadcasted_iota(jnp.int32, (128, 1, 1), 0).astype(jnp.float32) + c0
        M = assign[None, :, :] == C
        cnt = _r3(jnp.where(M, 1.0, 0.0), jnp.sum)
        mc = _r3(jnp.where(M, CONF[None], -INF), jnp.max)
        g = _r3(jnp.where(M & (CONF[None] == mc), IDX[None], INF), jnp.min)
        loc = _r3(jnp.where(M & (IDX[None] < g), 1.0, 0.0), jnp.sum)
        repr_ = jnp.where(cnt == 1.0, g, loc)
        valid = (cnt > 0.0) & (C < float(N_BOX))
        hits = (IDX[None] == repr_) & valid
        contrib = jnp.max(jnp.where(hits, 1.0, 0.0), axis=0)
        return jnp.maximum(maskacc, contrib)

    maskacc = lax.fori_loop(0, 8, cblock, jnp.zeros((8, 128), jnp.float32))
    out_ref[:, :] = maskacc


def kernel(bboxes_cxcywh, conf):
    coords = jnp.transpose(bboxes_cxcywh).astype(jnp.float32)
    coords = jnp.pad(coords, ((0, 0), (0, N_PAD - N_BOX)))
    coords_vec = coords.reshape(4, 8, 128)
    rep = jnp.repeat(coords, 8, axis=1).reshape(4, 8 * N_PAD, 1)
    ii = jnp.repeat(
        jnp.arange(N_PAD, dtype=jnp.float32), 8
    ).reshape(8 * N_PAD, 1)
    confp = jnp.pad(conf.astype(jnp.float32), (0, N_PAD - N_BOX)).reshape(8, 128)
    out = pl.pallas_call(
        _body,
        in_specs=[
            pl.BlockSpec(memory_space=pltpu.VMEM),
            pl.BlockSpec(memory_space=pltpu.VMEM),
            pl.BlockSpec(memory_space=pltpu.VMEM),
            pl.BlockSpec(memory_space=pltpu.VMEM),
        ],
        out_specs=pl.BlockSpec(memory_space=pltpu.VMEM),
        out_shape=jax.ShapeDtypeStruct((8, 128), jnp.float32),
        scratch_shapes=[pltpu.VMEM((N_PAD, 8, 128), jnp.float32)],
    )(coords_vec, rep, ii, confp)
    return out.reshape(N_PAD)[:N_BOX] > 0.5


# speedup vs baseline: 3.4178x; 2.2687x over previous
import jax
import jax.numpy as jnp
from jax import lax
from jax.experimental import pallas as pl
from jax.experimental.pallas import tpu as pltpu

N_BOX = 1000
N_PAD = 1024
INF = 1e9
IOU_T = 0.1


def _shift_lanes(v, k):
    return jnp.concatenate(
        [jnp.full((8, k), INF, jnp.float32), v[:, : 128 - k]], axis=1
    )


def _shift_subs(v, k):
    return jnp.concatenate(
        [jnp.full((k, 1), INF, jnp.float32), v[: 8 - k, :]], axis=0
    )


def _body(cv_ref, rep_ref, ii_ref, conf_ref, out_ref, adj_ref):
    cx, cy, w, h = cv_ref[0], cv_ref[1], cv_ref[2], cv_ref[3]
    X1 = cx - 0.5 * w
    Y1 = cy - 0.5 * h
    X2 = cx + 0.5 * w
    Y2 = cy + 0.5 * h
    AREA = w * h
    ROW = lax.broadcasted_iota(jnp.int32, (8, 128), 0)
    LANE = lax.broadcasted_iota(jnp.int32, (8, 128), 1)
    IDX = (ROW * 128 + LANE).astype(jnp.float32)
    CONF = conf_ref[:]

    def til(v):
        return jnp.broadcast_to(v[None], (64, 8, 128)).reshape(512, 128)

    X1T, Y1T, X2T, Y2T, AREAT, JT = (til(v) for v in (X1, Y1, X2, Y2, AREA, IDX))

    def ablock(bi, carry):
        base = bi * 512
        cxa = rep_ref[0, pl.ds(base, 512), :]
        cya = rep_ref[1, pl.ds(base, 512), :]
        wa = rep_ref[2, pl.ds(base, 512), :]
        ha = rep_ref[3, pl.ds(base, 512), :]
        iia = ii_ref[pl.ds(base, 512), :]
        x1a = cxa - 0.5 * wa
        y1a = cya - 0.5 * ha
        x2a = x1a + wa
        y2a = y1a + ha
        iw = jnp.maximum(jnp.minimum(x2a, X2T) - jnp.maximum(x1a, X1T), 0.0)
        ih = jnp.maximum(jnp.minimum(y2a, Y2T) - jnp.maximum(y1a, Y1T), 0.0)
        inter = iw * ih
        union = wa * ha + AREAT - inter
        adjb = (inter > IOU_T * union) & (JT > iia)
        val = jnp.where(JT == iia, 2.0, jnp.where(adjb, 1.0, 0.0))
        adj_ref[pl.ds(bi * 64, 64)] = val.reshape(64, 8, 128)
        return carry

    lax.fori_loop(0, 16, ablock, 0)

    def rowbody(i, assign):
        row = adj_ref[i]
        maskj = row == 1.0
        vfull = jnp.where(row >= 1.0, assign, INF)
        p = vfull
        for ks in (
            (1, 2, 3, 4, 5, 6, 7),
            (8, 16, 24, 32, 40, 48, 56, 64, 72, 80, 88, 96, 104, 112, 120),
        ):
            vals = [p] + [_shift_lanes(p, k) for k in ks]
            while len(vals) > 1:
                vals = [
                    jnp.minimum(vals[j], vals[j + 1]) if j + 1 < len(vals) else vals[j]
                    for j in range(0, len(vals), 2)
                ]
            p = vals[0]
        rowmin = jnp.min(vfull, axis=1, keepdims=True)
        t = _shift_subs(rowmin, 1)
        for k in (1, 2, 4):
            t = jnp.minimum(t, _shift_subs(t, k))
        p = jnp.minimum(p, t)
        totb = jnp.broadcast_to(
            jnp.minimum(rowmin[7:8, :], t[7:8, :]), (8, 128)
        )
        assign = jnp.where(maskj, p, assign)
        return jnp.where(row == 2.0, totb, assign)

    assign = lax.fori_loop(0, N_BOX, rowbody, IDX, unroll=False)

    def _r3(x, op):
        return op(op(x, axis=2, keepdims=True), axis=1, keepdims=True)

    def cblock(cb, maskacc):
        c0 = (cb * 128).astype(jnp.float32)
        C = lax.broadcasted_iota(jnp.int32, (128, 1, 1), 0).astype(jnp.float32) + c0
        M = assign[None, :, :] == C
        cnt = _r3(jnp.where(M, 1.0, 0.0), jnp.sum)
        mc = _r3(jnp.where(M, CONF[None], -INF), jnp.max)
        g = _r3(jnp.where(M & (CONF[None] == mc), IDX[None], INF), jnp.min)
        loc = _r3(jnp.where(M & (IDX[None] < g), 1.0, 0.0), jnp.sum)
        repr_ = jnp.where(cnt == 1.0, g, loc)
        valid = (cnt > 0.0) & (C < float(N_BOX))
        hits = (IDX[None] == repr_) & valid
        contrib = jnp.max(jnp.where(hits, 1.0, 0.0), axis=0)
        return jnp.maximum(maskacc, contrib)

    maskacc = lax.fori_loop(0, 8, cblock, jnp.zeros((8, 128), jnp.float32))
    out_ref[:, :] = maskacc


def kernel(bboxes_cxcywh, conf):
    coords = jnp.transpose(bboxes_cxcywh).astype(jnp.float32)
    coords = jnp.pad(coords, ((0, 0), (0, N_PAD - N_BOX)))
    coords_vec = coords.reshape(4, 8, 128)
    rep = jnp.repeat(coords, 8, axis=1).reshape(4, 8 * N_PAD, 1)
    ii = jnp.repeat(
        jnp.arange(N_PAD, dtype=jnp.float32), 8
    ).reshape(8 * N_PAD, 1)
    confp = jnp.pad(conf.astype(jnp.float32), (0, N_PAD - N_BOX)).reshape(8, 128)
    out = pl.pallas_call(
        _body,
        in_specs=[
            pl.BlockSpec(memory_space=pltpu.VMEM),
            pl.BlockSpec(memory_space=pltpu.VMEM),
            pl.BlockSpec(memory_space=pltpu.VMEM),
            pl.BlockSpec(memory_space=pltpu.VMEM),
        ],
        out_specs=pl.BlockSpec(memory_space=pltpu.VMEM),
        out_shape=jax.ShapeDtypeStruct((8, 128), jnp.float32),
        scratch_shapes=[pltpu.VMEM((N_PAD, 8, 128), jnp.float32)],
    )(coords_vec, rep, ii, confp)
    return out.reshape(N_PAD)[:N_BOX] > 0.5


# speedup vs baseline: 3.5351x; 1.0343x over previous
import jax
import jax.numpy as jnp
from jax import lax
from jax.experimental import pallas as pl
from jax.experimental.pallas import tpu as pltpu

N_BOX = 1000
N_PAD = 1024
INF = 1e9
IOU_T = 0.1


def _shift_lanes(v, k):
    return jnp.concatenate(
        [jnp.full((8, k), INF, jnp.float32), v[:, : 128 - k]], axis=1
    )


def _shift_subs(v, k):
    return jnp.concatenate(
        [jnp.full((k, 1), INF, jnp.float32), v[: 8 - k, :]], axis=0
    )


def _body(cv_ref, rep_ref, ii_ref, conf_ref, out_ref, adj_ref):
    cx, cy, w, h = cv_ref[0], cv_ref[1], cv_ref[2], cv_ref[3]
    X1 = cx - 0.5 * w
    Y1 = cy - 0.5 * h
    X2 = cx + 0.5 * w
    Y2 = cy + 0.5 * h
    AREA = w * h
    ROW = lax.broadcasted_iota(jnp.int32, (8, 128), 0)
    LANE = lax.broadcasted_iota(jnp.int32, (8, 128), 1)
    IDX = (ROW * 128 + LANE).astype(jnp.float32)
    CONF = conf_ref[:]

    def til(v):
        return jnp.broadcast_to(v[None], (64, 8, 128)).reshape(512, 128)

    X1T, Y1T, X2T, Y2T, AREAT, JT = (til(v) for v in (X1, Y1, X2, Y2, AREA, IDX))

    def ablock(bi, carry):
        base = bi * 512
        cxa = rep_ref[0, pl.ds(base, 512), :]
        cya = rep_ref[1, pl.ds(base, 512), :]
        wa = rep_ref[2, pl.ds(base, 512), :]
        ha = rep_ref[3, pl.ds(base, 512), :]
        iia = ii_ref[pl.ds(base, 512), :]
        x1a = cxa - 0.5 * wa
        y1a = cya - 0.5 * ha
        x2a = x1a + wa
        y2a = y1a + ha
        iw = jnp.maximum(jnp.minimum(x2a, X2T) - jnp.maximum(x1a, X1T), 0.0)
        ih = jnp.maximum(jnp.minimum(y2a, Y2T) - jnp.maximum(y1a, Y1T), 0.0)
        inter = iw * ih
        union = wa * ha + AREAT - inter
        adjb = (inter > IOU_T * union) & (JT > iia)
        val = jnp.where(JT == iia, 2.0, jnp.where(adjb, 1.0, 0.0))
        adj_ref[pl.ds(bi * 64, 64)] = val.reshape(64, 8, 128)
        return carry

    lax.fori_loop(0, 16, ablock, 0)

    def rowbody(i, assign):
        row = adj_ref[i]
        maskj = row == 1.0
        vfull = jnp.where(row >= 1.0, assign, INF)
        p = vfull
        for ks in (
            (1, 2, 3, 4, 5, 6, 7),
            (8, 16, 24, 32, 40, 48, 56, 64, 72, 80, 88, 96, 104, 112, 120),
        ):
            vals = [p] + [_shift_lanes(p, k) for k in ks]
            while len(vals) > 1:
                vals = [
                    jnp.minimum(vals[j], vals[j + 1]) if j + 1 < len(vals) else vals[j]
                    for j in range(0, len(vals), 2)
                ]
            p = vals[0]
        rowmin = jnp.min(vfull, axis=1, keepdims=True)
        t = _shift_subs(rowmin, 1)
        for k in (1, 2, 4):
            t = jnp.minimum(t, _shift_subs(t, k))
        p = jnp.minimum(p, t)
        totb = jnp.broadcast_to(
            jnp.minimum(rowmin[7:8, :], t[7:8, :]), (8, 128)
        )
        assign = jnp.where(maskj, p, assign)
        return jnp.where(row == 2.0, totb, assign)

    assign = lax.fori_loop(0, N_BOX, rowbody, IDX, unroll=2)

    def _r3(x, op):
        return op(op(x, axis=2, keepdims=True), axis=1, keepdims=True)

    def cblock(cb, maskacc):
        c0 = (cb * 128).astype(jnp.float32)
        C = lax.broadcasted_iota(jnp.int32, (128, 1, 1), 0).astype(jnp.float32) + c0
        M = assign[None, :, :] == C
        cnt = _r3(jnp.where(M, 1.0, 0.0), jnp.sum)
        mc = _r3(jnp.where(M, CONF[None], -INF), jnp.max)
        g = _r3(jnp.where(M & (CONF[None] == mc), IDX[None], INF), jnp.min)
        loc = _r3(jnp.where(M & (IDX[None] < g), 1.0, 0.0), jnp.sum)
        repr_ = jnp.where(cnt == 1.0, g, loc)
        valid = (cnt > 0.0) & (C < float(N_BOX))
        hits = (IDX[None] == repr_) & valid
        contrib = jnp.max(jnp.where(hits, 1.0, 0.0), axis=0)
        return jnp.maximum(maskacc, contrib)

    maskacc = lax.fori_loop(0, 8, cblock, jnp.zeros((8, 128), jnp.float32))
    out_ref[:, :] = maskacc


def kernel(bboxes_cxcywh, conf):
    coords = jnp.transpose(bboxes_cxcywh).astype(jnp.float32)
    coords = jnp.pad(coords, ((0, 0), (0, N_PAD - N_BOX)))
    coords_vec = coords.reshape(4, 8, 128)
    rep = jnp.repeat(coords, 8, axis=1).reshape(4, 8 * N_PAD, 1)
    ii = jnp.repeat(
        jnp.arange(N_PAD, dtype=jnp.float32), 8
    ).reshape(8 * N_PAD, 1)
    confp = jnp.pad(conf.astype(jnp.float32), (0, N_PAD - N_BOX)).reshape(8, 128)
    out = pl.pallas_call(
        _body,
        in_specs=[
            pl.BlockSpec(memory_space=pltpu.VMEM),
            pl.BlockSpec(memory_space=pltpu.VMEM),
            pl.BlockSpec(memory_space=pltpu.VMEM),
            pl.BlockSpec(memory_space=pltpu.VMEM),
        ],
        out_specs=pl.BlockSpec(memory_space=pltpu.VMEM),
        out_shape=jax.ShapeDtypeStruct((8, 128), jnp.float32),
        scratch_shapes=[pltpu.VMEM((N_PAD, 8, 128), jnp.float32)],
    )(coords_vec, rep, ii, confp)
    return out.reshape(N_PAD)[:N_BOX] > 0.5
